# Initial kernel scaffold; baseline (speedup 1.0000x reference)
import jax
import jax.numpy as jnp
from jax.experimental import pallas as pl


def kernel(x, edge_index, edge_attr, batch, W1, as1, ad1, b1, g1, be1, W2, as2, ad2, b2, g2, be2, W3, as3, ad3, b3, g3, be3, Wl1, bl1, Wl2, bl2, Wl3, bl3):
    raise NotImplementedError("write your pallas kernel here")


# speedup vs baseline: 1.6808x; 1.6808x over previous
import functools

import jax
import jax.numpy as jnp
from jax import lax
from jax.experimental import pallas as pl
from jax.experimental.pallas import tpu as pltpu

N = 10000
E = 160000
H = 3
C = 128
NH = H * C
G = 64

_ROWS = 1000


def _mm_alpha_body(x_ref, w_ref, a_ref, h_ref, al_ref):
    h = jnp.dot(x_ref[...], w_ref[...], preferred_element_type=jnp.float32)
    h_ref[...] = h
    hr = h.reshape(_ROWS, H, C)
    asrc = (hr * a_ref[0].reshape(1, H, C)).sum(-1)
    adst = (hr * a_ref[1].reshape(1, H, C)).sum(-1)
    al_ref[...] = jnp.concatenate(
        [asrc, adst, jnp.zeros((_ROWS, 2), jnp.float32)], axis=1)


def _mm_alpha(x, w, a_src, a_dst):
    f_in = x.shape[1]
    a2 = jnp.stack([a_src.reshape(NH), a_dst.reshape(NH)])
    return pl.pallas_call(
        _mm_alpha_body,
        grid=(N // _ROWS,),
        in_specs=[
            pl.BlockSpec((_ROWS, f_in), lambda i: (i, 0)),
            pl.BlockSpec((f_in, NH), lambda i: (0, 0)),
            pl.BlockSpec((2, NH), lambda i: (0, 0)),
        ],
        out_specs=[
            pl.BlockSpec((_ROWS, NH), lambda i: (i, 0)),
            pl.BlockSpec((_ROWS, 8), lambda i: (i, 0)),
        ],
        out_shape=[
            jax.ShapeDtypeStruct((N, NH), jnp.float32),
            jax.ShapeDtypeStruct((N, 8), jnp.float32),
        ],
    )(x, w, a2)


def _post_body(acc_ref, ssum_ref, h_ref, al_ref, b_ref, g_ref, be_ref,
               out_ref, new_ref):
    al = al_ref[...]
    l_self = al[:, 0:H] + al[:, H:2 * H]
    l_self = jnp.where(l_self >= 0, l_self, 0.2 * l_self)
    ex_self = jnp.exp(l_self)
    h = h_ref[...].reshape(_ROWS, H, C)
    acc = acc_ref[...].reshape(_ROWS, H, C)
    ssum = ssum_ref[...][:, 0:H]
    num = acc + h * ex_self[:, :, None]
    den = ssum + ex_self + 1e-16
    o = (num / den[:, :, None]).reshape(_ROWS, NH) + b_ref[...].reshape(1, NH)
    o = jnp.maximum(o, 0.0)
    o = g_ref[...].reshape(1, NH) * (o / jnp.sqrt(1.0 + 1e-5)) \
        + be_ref[...].reshape(1, NH)
    out_ref[...] = o
    new_ref[...] = o


def _post(acc, ssum, h, al, b, g, be):
    return pl.pallas_call(
        _post_body,
        grid=(N // _ROWS,),
        in_specs=[
            pl.BlockSpec((_ROWS, NH), lambda i: (i, 0)),
            pl.BlockSpec((_ROWS, 8), lambda i: (i, 0)),
            pl.BlockSpec((_ROWS, NH), lambda i: (i, 0)),
            pl.BlockSpec((_ROWS, 8), lambda i: (i, 0)),
            pl.BlockSpec((NH,), lambda i: (0,)),
            pl.BlockSpec((NH,), lambda i: (0,)),
            pl.BlockSpec((NH,), lambda i: (0,)),
        ],
        out_specs=[
            pl.BlockSpec((_ROWS, NH), lambda i: (i, 0)),
            pl.BlockSpec((_ROWS, NH), lambda i: (i, 0)),
        ],
        out_shape=[
            jax.ShapeDtypeStruct((N, NH), jnp.float32),
            jax.ShapeDtypeStruct((N, NH), jnp.float32),
        ],
    )(acc, ssum, h, al, b, g, be)


def _mlp_body(z_ref, w1_ref, b1_ref, w2_ref, b2_ref, w3_ref, b3_ref, o_ref):
    z = jnp.maximum(jnp.dot(z_ref[...], w1_ref[...],
                            preferred_element_type=jnp.float32)
                    + b1_ref[...].reshape(1, -1), 0.0)
    z = jnp.maximum(jnp.dot(z, w2_ref[...],
                            preferred_element_type=jnp.float32)
                    + b2_ref[...].reshape(1, -1), 0.0)
    o = jnp.dot(z, w3_ref[...], preferred_element_type=jnp.float32) \
        + b3_ref[...].reshape(1, -1)
    o_ref[...] = o[:, 0:2]


def _mlp(z, wl1, bl1, wl2, bl2, wl3, bl3):
    return pl.pallas_call(
        _mlp_body,
        out_shape=jax.ShapeDtypeStruct((G, 2), jnp.float32),
    )(z, wl1, bl1, wl2, bl2, wl3, bl3)


def _edge_pass_xla(h, al, src, dst):
    l = al[src, 0:H] + al[dst, H:2 * H]
    l = jnp.where(l >= 0, l, 0.2 * l)
    ex = jnp.exp(l)
    msg = h[src].reshape(E, H, C) * ex[:, :, None]
    acc = jax.ops.segment_sum(msg.reshape(E, NH), dst, num_segments=N)
    ssum = jax.ops.segment_sum(ex, dst, num_segments=N)
    ssum = jnp.concatenate([ssum, jnp.zeros((N, 5), jnp.float32)], axis=1)
    return acc, ssum


def _pool_xla(x, batch):
    s = jax.ops.segment_sum(x, batch, num_segments=G)
    cnt = jax.ops.segment_sum(jnp.ones((N,), jnp.float32), batch,
                              num_segments=G)
    mean = s / jnp.maximum(cnt, 1.0)[:, None]
    mx = jax.ops.segment_max(x, batch, num_segments=G)
    mx = jnp.where(jnp.isfinite(mx), mx, 0.0)
    return jnp.concatenate([mean, mx], axis=1)


def kernel(x, edge_index, edge_attr, batch, W1, as1, ad1, b1, g1, be1,
           W2, as2, ad2, b2, g2, be2, W3, as3, ad3, b3, g3, be3,
           Wl1, bl1, Wl2, bl2, Wl3, bl3):
    src = edge_index[0]
    dst = edge_index[1]

    z = None
    h_in = x
    for (W, a_s, a_d, b, g, be) in (
            (W1, as1, ad1, b1, g1, be1),
            (W2, as2, ad2, b2, g2, be2),
            (W3, as3, ad3, b3, g3, be3)):
        h, al = _mm_alpha(h_in, W, a_s, a_d)
        acc, ssum = _edge_pass_xla(h, al, src, dst)
        out, h_in = _post(acc, ssum, h, al, b, g, be)
        p = _pool_xla(out, batch)
        z = p if z is None else z + p

    return _mlp(z, Wl1, bl1, Wl2, bl2, Wl3, bl3)


# speedup vs baseline: 15.7117x; 9.3478x over previous
import functools

import jax
import jax.numpy as jnp
from jax import lax
from jax.experimental import pallas as pl
from jax.experimental.pallas import tpu as pltpu
from jax.experimental.pallas import tpu_sc as plsc

N = 10000
E = 160000
H = 3
C = 128
NH = H * C
G = 64

_ROWS = 1000
_EW = 400

_SC_NC = 2
_SC_NS = 16
_L = 16
_NQ = 4
_QROWS = N // _NQ
_ACC_ROWS = 2560
_ALDQ = 7504
_EPT = E // _SC_NS
_SEG = 2000
_NSTRIPE = _EPT // _SEG


def _mm_alpha_body(x_ref, w_ref, a_ref, h_ref, al_ref, he_ref):
    h = jnp.dot(x_ref[...], w_ref[...], preferred_element_type=jnp.float32)
    h_ref[...] = h
    hr = h.reshape(_ROWS, H, C)
    asrc = (hr * a_ref[0].reshape(1, H, C)).sum(-1)
    adst = (hr * a_ref[1].reshape(1, H, C)).sum(-1)
    al_ref[...] = jnp.concatenate(
        [asrc, adst, jnp.zeros((_ROWS, 2), jnp.float32)], axis=1)
    he_ref[...] = jnp.concatenate(
        [h, jnp.ones((_ROWS, 3), jnp.float32),
         jnp.zeros((_ROWS, _EW - NH - 3), jnp.float32)], axis=1)


def _mm_alpha(x, w, a_src, a_dst):
    f_in = x.shape[1]
    a2 = jnp.stack([a_src.reshape(NH), a_dst.reshape(NH)])
    return pl.pallas_call(
        _mm_alpha_body,
        grid=(N // _ROWS,),
        in_specs=[
            pl.BlockSpec((_ROWS, f_in), lambda i: (i, 0)),
            pl.BlockSpec((f_in, NH), lambda i: (0, 0)),
            pl.BlockSpec((2, NH), lambda i: (0, 0)),
        ],
        out_specs=[
            pl.BlockSpec((_ROWS, NH), lambda i: (i, 0)),
            pl.BlockSpec((_ROWS, 8), lambda i: (i, 0)),
            pl.BlockSpec((_ROWS, _EW), lambda i: (i, 0)),
        ],
        out_shape=[
            jax.ShapeDtypeStruct((N, NH), jnp.float32),
            jax.ShapeDtypeStruct((N, 8), jnp.float32),
            jax.ShapeDtypeStruct((N, _EW), jnp.float32),
        ],
    )(x, w, a2)


def _edge_body(src_hbm, dst_hbm, als_hbm, aldq_hbm, hext_hbm, out_hbm,
               als_v, ald_v, sstripe, dstripe, srcb, dstb, eb0, eb1, eb2,
               rows, accum, sem):
    cid = lax.axis_index("c")
    tid = lax.axis_index("s")
    iota = jnp.arange(_L, dtype=jnp.int32)
    ones = jnp.ones((_L,), jnp.float32)
    zeros_f = jnp.zeros((_L,), jnp.float32)
    zeros_i = jnp.zeros((_L,), jnp.int32)

    pltpu.sync_copy(als_hbm, als_v)

    for j in range(_L):
        for c in range(_EW // _L):
            rows[j, pl.ds(c * _L, _L)] = zeros_f

    for p in range(2):
        q = 2 * p + cid
        lo = q * _QROWS
        hi = lo + _QROWS

        pltpu.sync_copy(aldq_hbm.at[q], ald_v)

        def _zero_body(i, _):
            pltpu.sync_copy(rows,
                            accum.at[pl.ds(tid * (_ACC_ROWS // _SC_NS)
                                           + i * _L, _L)])
            return 0

        lax.fori_loop(0, _ACC_ROWS // _SC_NS // _L, _zero_body, 0)
        plsc.subcore_barrier()

        def _stripe_body(st, _):
            base = tid * _EPT + st * _SEG
            pltpu.sync_copy(src_hbm.at[pl.ds(base, _SEG)], sstripe)
            pltpu.sync_copy(dst_hbm.at[pl.ds(base, _SEG)], dstripe)

            def _scan_body(v, k):
                s_vec = sstripe[pl.ds(v * _L, _L)]
                d_vec = dstripe[pl.ds(v * _L, _L)]
                inr = (d_vec >= lo) & (d_vec < hi)
                d_loc = jnp.where(inr, d_vec - lo, 0)
                idx_s = s_vec * 3
                idx_d = d_loc * 3
                ex = []
                for h in range(H):
                    l = (plsc.load_gather(als_v, [idx_s + h])
                         + plsc.load_gather(ald_v, [idx_d + h]))
                    l = jnp.where(l >= 0, l, 0.2 * l)
                    ex.append(jnp.exp(l))
                plsc.store_compressed(srcb.at[pl.ds(k, _L)], s_vec,
                                      mask=inr)
                plsc.store_compressed(dstb.at[pl.ds(k, _L)], d_loc,
                                      mask=inr)
                plsc.store_compressed(eb0.at[pl.ds(k, _L)], ex[0], mask=inr)
                plsc.store_compressed(eb1.at[pl.ds(k, _L)], ex[1], mask=inr)
                plsc.store_compressed(eb2.at[pl.ds(k, _L)], ex[2], mask=inr)
                return k + jnp.sum(inr.astype(jnp.int32))

            k = lax.fori_loop(0, _SEG // _L, _scan_body, jnp.int32(0))

            srcb[pl.ds(k, _L)] = zeros_i
            dstb[pl.ds(k, _L)] = jnp.full((_L,), _QROWS, jnp.int32) + tid
            eb0[pl.ds(k, _L)] = zeros_f
            eb1[pl.ds(k, _L)] = zeros_f
            eb2[pl.ds(k, _L)] = zeros_f
            nproc = (k + (_L - 1)) >> 4

            def _proc_body(i, _):
                sv = srcb[pl.ds(i * _L, _L)]
                dv = dstb[pl.ds(i * _L, _L)]
                pltpu.async_copy(hext_hbm.at[sv], rows, sem).wait()
                for j in range(_L):
                    bidx = jnp.full((_L,), i * _L + j, jnp.int32)
                    e0 = plsc.load_gather(eb0, [bidx])
                    e1 = plsc.load_gather(eb1, [bidx])
                    e2 = plsc.load_gather(eb2, [bidx])
                    for h, ev in ((0, e0), (1, e1), (2, e2)):
                        for cc in range(C // _L):
                            off = h * C + cc * _L
                            rows[j, pl.ds(off, _L)] = (
                                rows[j, pl.ds(off, _L)] * ev)
                    tail = jnp.where(iota == 0, e0,
                                     jnp.where(iota == 1, e1,
                                               jnp.where(iota == 2, e2,
                                                         ones)))
                    rows[j, pl.ds(NH, _L)] = rows[j, pl.ds(NH, _L)] * tail
                pltpu.sync_copy(rows, accum.at[dv], add=True)
                return 0

            lax.fori_loop(0, nproc, _proc_body, 0)
            return 0

        lax.fori_loop(0, _NSTRIPE, _stripe_body, 0)
        plsc.subcore_barrier()

        @pl.when(tid < _SC_NS - 1)
        def _():
            pltpu.sync_copy(accum.at[pl.ds(tid * 152, 152)],
                            out_hbm.at[pl.ds(lo + tid * 152, 152)])

        @pl.when(tid == _SC_NS - 1)
        def _():
            pltpu.sync_copy(accum.at[pl.ds((_SC_NS - 1) * 152, 220)],
                            out_hbm.at[pl.ds(lo + (_SC_NS - 1) * 152, 220)])

        if p == 0:
            for j in range(_L):
                for c in range(_EW // _L):
                    rows[j, pl.ds(c * _L, _L)] = zeros_f
            plsc.subcore_barrier()


@functools.partial(
    pl.kernel,
    out_type=jax.ShapeDtypeStruct((N, _EW), jnp.float32),
    mesh=plsc.VectorSubcoreMesh(core_axis_name="c", subcore_axis_name="s"),
    compiler_params=pltpu.CompilerParams(use_tc_tiling_on_sc=False,
                                         needs_layout_passes=False),
    scratch_types=[
        pltpu.VMEM((N * H,), jnp.float32),
        pltpu.VMEM((_ALDQ,), jnp.float32),
        pltpu.VMEM((_SEG,), jnp.int32),
        pltpu.VMEM((_SEG,), jnp.int32),
        pltpu.VMEM((_SEG + _L,), jnp.int32),
        pltpu.VMEM((_SEG + _L,), jnp.int32),
        pltpu.VMEM((_SEG + _L,), jnp.float32),
        pltpu.VMEM((_SEG + _L,), jnp.float32),
        pltpu.VMEM((_SEG + _L,), jnp.float32),
        pltpu.VMEM((_L, _EW), jnp.float32),
        pltpu.VMEM_SHARED((_ACC_ROWS, _EW), jnp.float32),
        pltpu.SemaphoreType.DMA,
    ],
)
def _edge_pass_sc(src_hbm, dst_hbm, als_hbm, aldq_hbm, hext_hbm, out_hbm,
                  *rest):
    _edge_body(src_hbm, dst_hbm, als_hbm, aldq_hbm, hext_hbm, out_hbm, *rest)


def _post_body(acc_ref, h_ref, al_ref, b_ref, g_ref, be_ref, out_ref):
    al = al_ref[...]
    l_self = al[:, 0:H] + al[:, H:2 * H]
    l_self = jnp.where(l_self >= 0, l_self, 0.2 * l_self)
    ex_self = jnp.exp(l_self)
    h = h_ref[...].reshape(_ROWS, H, C)
    acc_full = acc_ref[...]
    acc = acc_full[:, 0:NH].reshape(_ROWS, H, C)
    ssum = acc_full[:, NH:NH + H]
    num = acc + h * ex_self[:, :, None]
    den = ssum + ex_self + 1e-16
    o = (num / den[:, :, None]).reshape(_ROWS, NH) + b_ref[...].reshape(1, NH)
    o = jnp.maximum(o, 0.0)
    o = g_ref[...].reshape(1, NH) * (o / jnp.sqrt(1.0 + 1e-5)) \
        + be_ref[...].reshape(1, NH)
    out_ref[...] = o


def _post(acc, h, al, b, g, be):
    return pl.pallas_call(
        _post_body,
        grid=(N // _ROWS,),
        in_specs=[
            pl.BlockSpec((_ROWS, _EW), lambda i: (i, 0)),
            pl.BlockSpec((_ROWS, NH), lambda i: (i, 0)),
            pl.BlockSpec((_ROWS, 8), lambda i: (i, 0)),
            pl.BlockSpec((NH,), lambda i: (0,)),
            pl.BlockSpec((NH,), lambda i: (0,)),
            pl.BlockSpec((NH,), lambda i: (0,)),
        ],
        out_specs=pl.BlockSpec((_ROWS, NH), lambda i: (i, 0)),
        out_shape=jax.ShapeDtypeStruct((N, NH), jnp.float32),
    )(acc, h, al, b, g, be)


def _pool_xla(x, batch):
    s = jax.ops.segment_sum(x, batch, num_segments=G)
    cnt = jax.ops.segment_sum(jnp.ones((N,), jnp.float32), batch,
                              num_segments=G)
    mean = s / jnp.maximum(cnt, 1.0)[:, None]
    mx = jax.ops.segment_max(x, batch, num_segments=G)
    mx = jnp.where(jnp.isfinite(mx), mx, 0.0)
    return jnp.concatenate([mean, mx], axis=1)


def _mlp_body(z1_ref, z2_ref, z3_ref, w1_ref, b1_ref, w2_ref, b2_ref,
              w3_ref, b3_ref, o_ref):
    z = z1_ref[...] + z2_ref[...] + z3_ref[...]
    z = jnp.maximum(jnp.dot(z, w1_ref[...],
                            preferred_element_type=jnp.float32)
                    + b1_ref[...].reshape(1, -1), 0.0)
    z = jnp.maximum(jnp.dot(z, w2_ref[...],
                            preferred_element_type=jnp.float32)
                    + b2_ref[...].reshape(1, -1), 0.0)
    o = jnp.dot(z, w3_ref[...], preferred_element_type=jnp.float32) \
        + b3_ref[...].reshape(1, -1)
    o_ref[...] = o[:, 0:2]


def _mlp(z1, z2, z3, wl1, bl1, wl2, bl2, wl3, bl3):
    return pl.pallas_call(
        _mlp_body,
        out_shape=jax.ShapeDtypeStruct((G, 2), jnp.float32),
    )(z1, z2, z3, wl1, bl1, wl2, bl2, wl3, bl3)


def kernel(x, edge_index, edge_attr, batch, W1, as1, ad1, b1, g1, be1,
           W2, as2, ad2, b2, g2, be2, W3, as3, ad3, b3, g3, be3,
           Wl1, bl1, Wl2, bl2, Wl3, bl3):
    src = edge_index[0]
    dst = edge_index[1]

    pools = []
    h_in = x
    for (W, a_s, a_d, b, g, be) in (
            (W1, as1, ad1, b1, g1, be1),
            (W2, as2, ad2, b2, g2, be2),
            (W3, as3, ad3, b3, g3, be3)):
        h, al, hext = _mm_alpha(h_in, W, a_s, a_d)
        als = al[:, 0:H].reshape(N * H)
        ald = al[:, H:2 * H].reshape(N * H)
        aldq = jnp.zeros((_NQ, _ALDQ), jnp.float32).at[:, 0:N * H // _NQ].set(
            ald.reshape(_NQ, N * H // _NQ))
        acc = _edge_pass_sc(src, dst, als, aldq, hext)
        h_in = _post(acc, h, al, b, g, be)
        pools.append(_pool_xla(h_in, batch))

    return _mlp(*pools, Wl1, bl1, Wl2, bl2, Wl3, bl3)


# speedup vs baseline: 21.0331x; 1.3387x over previous
import functools

import jax
import jax.numpy as jnp
from jax import lax
from jax.experimental import pallas as pl
from jax.experimental.pallas import tpu as pltpu
from jax.experimental.pallas import tpu_sc as plsc

N = 10000
E = 160000
H = 3
C = 128
NH = H * C
G = 64

_ROWS = 1000
_EW = 400

_SC_NC = 2
_SC_NS = 16
_L = 16
_NQ = 4
_QROWS = N // _NQ
_ACC_ROWS = 2560
_ALDQ = 7504
_EPT = E // _SC_NS
_SEG = 2000
_NSTRIPE = _EPT // _SEG


def _mm_alpha_body(x_ref, w_ref, a_ref, h_ref, al_ref, he_ref):
    h = jnp.dot(x_ref[...], w_ref[...], preferred_element_type=jnp.float32)
    h_ref[...] = h
    hr = h.reshape(_ROWS, H, C)
    asrc = (hr * a_ref[0].reshape(1, H, C)).sum(-1)
    adst = (hr * a_ref[1].reshape(1, H, C)).sum(-1)
    al_ref[...] = jnp.concatenate(
        [asrc, adst, jnp.zeros((_ROWS, 2), jnp.float32)], axis=1)
    he_ref[...] = jnp.concatenate(
        [h, jnp.ones((_ROWS, 3), jnp.float32),
         jnp.zeros((_ROWS, _EW - NH - 3), jnp.float32)], axis=1)


def _mm_alpha(x, w, a_src, a_dst):
    f_in = x.shape[1]
    a2 = jnp.stack([a_src.reshape(NH), a_dst.reshape(NH)])
    return pl.pallas_call(
        _mm_alpha_body,
        grid=(N // _ROWS,),
        in_specs=[
            pl.BlockSpec((_ROWS, f_in), lambda i: (i, 0)),
            pl.BlockSpec((f_in, NH), lambda i: (0, 0)),
            pl.BlockSpec((2, NH), lambda i: (0, 0)),
        ],
        out_specs=[
            pl.BlockSpec((_ROWS, NH), lambda i: (i, 0)),
            pl.BlockSpec((_ROWS, 8), lambda i: (i, 0)),
            pl.BlockSpec((_ROWS, _EW), lambda i: (i, 0)),
        ],
        out_shape=[
            jax.ShapeDtypeStruct((N, NH), jnp.float32),
            jax.ShapeDtypeStruct((N, 8), jnp.float32),
            jax.ShapeDtypeStruct((N, _EW), jnp.float32),
        ],
    )(x, w, a2)


def _edge_body(src_hbm, dst_hbm, als_hbm, aldq_hbm, hext_hbm, out_hbm,
               als_v, ald_v, sstripe, dstripe, srcb, dstb, eb0, eb1, eb2,
               rows, accum, sem):
    cid = lax.axis_index("c")
    tid = lax.axis_index("s")
    iota = jnp.arange(_L, dtype=jnp.int32)
    ones = jnp.ones((_L,), jnp.float32)
    zeros_f = jnp.zeros((_L,), jnp.float32)
    zeros_i = jnp.zeros((_L,), jnp.int32)

    pltpu.sync_copy(als_hbm, als_v)

    for j in range(_L):
        for c in range(_EW // _L):
            rows[j, pl.ds(c * _L, _L)] = zeros_f

    for p in range(2):
        q = 2 * p + cid
        lo = q * _QROWS
        hi = lo + _QROWS

        pltpu.sync_copy(aldq_hbm.at[q], ald_v)

        def _zero_body(i, _):
            pltpu.sync_copy(rows,
                            accum.at[pl.ds(tid * (_ACC_ROWS // _SC_NS)
                                           + i * _L, _L)])
            return 0

        lax.fori_loop(0, _ACC_ROWS // _SC_NS // _L, _zero_body, 0)
        plsc.subcore_barrier()

        def _stripe_body(st, _):
            base = tid * _EPT + st * _SEG
            pltpu.sync_copy(src_hbm.at[pl.ds(base, _SEG)], sstripe)
            pltpu.sync_copy(dst_hbm.at[pl.ds(base, _SEG)], dstripe)

            def _scan_body(v, k):
                s_vec = sstripe[pl.ds(v * _L, _L)]
                d_vec = dstripe[pl.ds(v * _L, _L)]
                inr = (d_vec >= lo) & (d_vec < hi)
                d_loc = jnp.where(inr, d_vec - lo, 0)
                idx_s = s_vec * 3
                idx_d = d_loc * 3
                ex = []
                for h in range(H):
                    l = (plsc.load_gather(als_v, [idx_s + h])
                         + plsc.load_gather(ald_v, [idx_d + h]))
                    l = jnp.where(l >= 0, l, 0.2 * l)
                    ex.append(jnp.exp(l))
                plsc.store_compressed(srcb.at[pl.ds(k, _L)], s_vec,
                                      mask=inr)
                plsc.store_compressed(dstb.at[pl.ds(k, _L)], d_loc,
                                      mask=inr)
                plsc.store_compressed(eb0.at[pl.ds(k, _L)], ex[0], mask=inr)
                plsc.store_compressed(eb1.at[pl.ds(k, _L)], ex[1], mask=inr)
                plsc.store_compressed(eb2.at[pl.ds(k, _L)], ex[2], mask=inr)
                return k + jnp.sum(inr.astype(jnp.int32))

            k = lax.fori_loop(0, _SEG // _L, _scan_body, jnp.int32(0))

            srcb[pl.ds(k, _L)] = zeros_i
            dstb[pl.ds(k, _L)] = jnp.full((_L,), _QROWS, jnp.int32) + tid
            eb0[pl.ds(k, _L)] = zeros_f
            eb1[pl.ds(k, _L)] = zeros_f
            eb2[pl.ds(k, _L)] = zeros_f
            nproc = (k + (_L - 1)) >> 4

            def _proc_body(i, _):
                sv = srcb[pl.ds(i * _L, _L)]
                dv = dstb[pl.ds(i * _L, _L)]
                pltpu.async_copy(hext_hbm.at[sv], rows, sem).wait()
                for j in range(_L):
                    bidx = jnp.full((_L,), i * _L + j, jnp.int32)
                    e0 = plsc.load_gather(eb0, [bidx])
                    e1 = plsc.load_gather(eb1, [bidx])
                    e2 = plsc.load_gather(eb2, [bidx])
                    for h, ev in ((0, e0), (1, e1), (2, e2)):
                        for cc in range(C // _L):
                            off = h * C + cc * _L
                            rows[j, pl.ds(off, _L)] = (
                                rows[j, pl.ds(off, _L)] * ev)
                    tail = jnp.where(iota == 0, e0,
                                     jnp.where(iota == 1, e1,
                                               jnp.where(iota == 2, e2,
                                                         ones)))
                    rows[j, pl.ds(NH, _L)] = rows[j, pl.ds(NH, _L)] * tail
                pltpu.sync_copy(rows, accum.at[dv], add=True)
                return 0

            lax.fori_loop(0, nproc, _proc_body, 0)
            return 0

        lax.fori_loop(0, _NSTRIPE, _stripe_body, 0)
        plsc.subcore_barrier()

        @pl.when(tid < _SC_NS - 1)
        def _():
            pltpu.sync_copy(accum.at[pl.ds(tid * 152, 152)],
                            out_hbm.at[pl.ds(lo + tid * 152, 152)])

        @pl.when(tid == _SC_NS - 1)
        def _():
            pltpu.sync_copy(accum.at[pl.ds((_SC_NS - 1) * 152, 220)],
                            out_hbm.at[pl.ds(lo + (_SC_NS - 1) * 152, 220)])

        if p == 0:
            for j in range(_L):
                for c in range(_EW // _L):
                    rows[j, pl.ds(c * _L, _L)] = zeros_f
            plsc.subcore_barrier()


@functools.cache
def _edge_pass_sc_fn():
    return pl.kernel(
        _edge_body,
        out_type=jax.ShapeDtypeStruct((N, _EW), jnp.float32),
        mesh=plsc.VectorSubcoreMesh(core_axis_name="c",
                                    subcore_axis_name="s",
                                    num_cores=_SC_NC, num_subcores=_SC_NS),
        compiler_params=pltpu.CompilerParams(use_tc_tiling_on_sc=False,
                                             needs_layout_passes=False),
        scratch_types=[
            pltpu.VMEM((N * H,), jnp.float32),
            pltpu.VMEM((_ALDQ,), jnp.float32),
            pltpu.VMEM((_SEG,), jnp.int32),
            pltpu.VMEM((_SEG,), jnp.int32),
            pltpu.VMEM((_SEG + _L,), jnp.int32),
            pltpu.VMEM((_SEG + _L,), jnp.int32),
            pltpu.VMEM((_SEG + _L,), jnp.float32),
            pltpu.VMEM((_SEG + _L,), jnp.float32),
            pltpu.VMEM((_SEG + _L,), jnp.float32),
            pltpu.VMEM((_L, _EW), jnp.float32),
            pltpu.VMEM_SHARED((_ACC_ROWS, _EW), jnp.float32),
            pltpu.SemaphoreType.DMA,
        ],
    )


def _edge_pass_sc(src, dst, als, aldq, hext):
    return _edge_pass_sc_fn()(src, dst, als, aldq, hext)


def _post_body(acc_ref, h_ref, al_ref, b_ref, g_ref, be_ref, out_ref):
    al = al_ref[...]
    l_self = al[:, 0:H] + al[:, H:2 * H]
    l_self = jnp.where(l_self >= 0, l_self, 0.2 * l_self)
    ex_self = jnp.exp(l_self)
    h = h_ref[...].reshape(_ROWS, H, C)
    acc_full = acc_ref[...]
    acc = acc_full[:, 0:NH].reshape(_ROWS, H, C)
    ssum = acc_full[:, NH:NH + H]
    num = acc + h * ex_self[:, :, None]
    den = ssum + ex_self + 1e-16
    o = (num / den[:, :, None]).reshape(_ROWS, NH) + b_ref[...].reshape(1, NH)
    o = jnp.maximum(o, 0.0)
    o = g_ref[...].reshape(1, NH) * (o / jnp.sqrt(1.0 + 1e-5)) \
        + be_ref[...].reshape(1, NH)
    out_ref[...] = o


def _post(acc, h, al, b, g, be):
    return pl.pallas_call(
        _post_body,
        grid=(N // _ROWS,),
        in_specs=[
            pl.BlockSpec((_ROWS, _EW), lambda i: (i, 0)),
            pl.BlockSpec((_ROWS, NH), lambda i: (i, 0)),
            pl.BlockSpec((_ROWS, 8), lambda i: (i, 0)),
            pl.BlockSpec((NH,), lambda i: (0,)),
            pl.BlockSpec((NH,), lambda i: (0,)),
            pl.BlockSpec((NH,), lambda i: (0,)),
        ],
        out_specs=pl.BlockSpec((_ROWS, NH), lambda i: (i, 0)),
        out_shape=jax.ShapeDtypeStruct((N, NH), jnp.float32),
    )(acc, h, al, b, g, be)


def _pool_body(x_ref, bt_ref, out_ref, accsum, accmax, acccnt):
    i = pl.program_id(0)

    @pl.when(i == 0)
    def _():
        accsum[...] = jnp.zeros((G, NH), jnp.float32)
        accmax[...] = jnp.full((G, NH), -3e38, jnp.float32)
        acccnt[...] = jnp.zeros((G, 128), jnp.float32)

    bt_col = bt_ref[...]
    x = x_ref[...]
    giota = lax.broadcasted_iota(jnp.int32, (1, G), 1)
    onehot = jnp.where(bt_col == giota, 1.0, 0.0)
    accsum[...] += lax.dot_general(onehot, x, (((0,), (0,)), ((), ())),
                                   preferred_element_type=jnp.float32)
    cnt_col = lax.dot_general(onehot, jnp.ones((_ROWS, 1), jnp.float32),
                              (((0,), (0,)), ((), ())),
                              preferred_element_type=jnp.float32)
    acccnt[...] += jnp.broadcast_to(cnt_col, (G, 128))
    glo = jnp.min(bt_col)
    ghi = jnp.max(bt_col)

    def _gmax(gk, _):
        @pl.when(gk <= ghi - glo)
        def _():
            g = glo + gk
            mrow = jnp.max(jnp.where(bt_col == g, x, -3e38), axis=0,
                           keepdims=True)
            cur = accmax[pl.ds(g, 1), :]
            accmax[pl.ds(g, 1), :] = jnp.maximum(cur, mrow)
        return 0

    lax.fori_loop(0, G, _gmax, 0)

    @pl.when(i == N // _ROWS - 1)
    def _():
        cnt = acccnt[...][:, 0:1]
        mean = accsum[...] / jnp.maximum(cnt, 1.0)
        mx = accmax[...]
        mx = jnp.where(mx > -1e38, mx, 0.0)
        out_ref[...] = jnp.concatenate([mean, mx], axis=1)


def _pool(x, batch_col):
    return pl.pallas_call(
        _pool_body,
        grid=(N // _ROWS,),
        in_specs=[
            pl.BlockSpec((_ROWS, NH), lambda i: (i, 0)),
            pl.BlockSpec((_ROWS, 1), lambda i: (i, 0)),
        ],
        out_specs=pl.BlockSpec((G, 2 * NH), lambda i: (0, 0)),
        out_shape=jax.ShapeDtypeStruct((G, 2 * NH), jnp.float32),
        scratch_shapes=[
            pltpu.VMEM((G, NH), jnp.float32),
            pltpu.VMEM((G, NH), jnp.float32),
            pltpu.VMEM((G, 128), jnp.float32),
        ],
    )(x, batch_col)


def _mlp_body(z1_ref, z2_ref, z3_ref, w1_ref, b1_ref, w2_ref, b2_ref,
              w3_ref, b3_ref, o_ref):
    z = z1_ref[...] + z2_ref[...] + z3_ref[...]
    z = jnp.maximum(jnp.dot(z, w1_ref[...],
                            preferred_element_type=jnp.float32)
                    + b1_ref[...].reshape(1, -1), 0.0)
    z = jnp.maximum(jnp.dot(z, w2_ref[...],
                            preferred_element_type=jnp.float32)
                    + b2_ref[...].reshape(1, -1), 0.0)
    o = jnp.dot(z, w3_ref[...], preferred_element_type=jnp.float32) \
        + b3_ref[...].reshape(1, -1)
    o_ref[...] = o[:, 0:2]


def _mlp(z1, z2, z3, wl1, bl1, wl2, bl2, wl3, bl3):
    return pl.pallas_call(
        _mlp_body,
        out_shape=jax.ShapeDtypeStruct((G, 2), jnp.float32),
    )(z1, z2, z3, wl1, bl1, wl2, bl2, wl3, bl3)


def kernel(x, edge_index, edge_attr, batch, W1, as1, ad1, b1, g1, be1,
           W2, as2, ad2, b2, g2, be2, W3, as3, ad3, b3, g3, be3,
           Wl1, bl1, Wl2, bl2, Wl3, bl3):
    src = edge_index[0]
    dst = edge_index[1]
    batch_col = batch.reshape(N, 1)

    pools = []
    h_in = x
    for (W, a_s, a_d, b, g, be) in (
            (W1, as1, ad1, b1, g1, be1),
            (W2, as2, ad2, b2, g2, be2),
            (W3, as3, ad3, b3, g3, be3)):
        h, al, hext = _mm_alpha(h_in, W, a_s, a_d)
        als = al[:, 0:H].reshape(N * H)
        ald = al[:, H:2 * H].reshape(N * H)
        aldq = jnp.zeros((_NQ, _ALDQ), jnp.float32).at[:, 0:N * H // _NQ].set(
            ald.reshape(_NQ, N * H // _NQ))
        acc = _edge_pass_sc(src, dst, als, aldq, hext)
        h_in = _post(acc, h, al, b, g, be)
        pools.append(_pool(h_in, batch_col))

    return _mlp(*pools, Wl1, bl1, Wl2, bl2, Wl3, bl3)
